# baseline (speedup 1.0000x reference)
import functools

import jax
import jax.numpy as jnp
from jax.experimental import pallas as pl
from jax.experimental.pallas import tpu as pltpu


def _lora_block(x_ref, tid_ref, a_ref, b_ref, out_ref, *, rank):
    n_cols = a_ref.shape[1]
    tb = x_ref.shape[0]
    tid = jnp.reshape(tid_ref[...], (tb, 1))
    col_expert = jax.lax.broadcasted_iota(jnp.int32, (tb, n_cols), 1) // rank
    xa = jnp.dot(x_ref[...], a_ref[...], preferred_element_type=jnp.float32)
    xa = jnp.where(tid == col_expert, xa, 0.0)
    out_ref[...] = jnp.dot(xa, b_ref[...], preferred_element_type=jnp.float32)


def kernel(x, task_id, lora_A, lora_B):
    T, in_dim = x.shape
    n_tasks, _, rank = lora_A.shape
    out_dim = lora_B.shape[2]
    er = n_tasks * rank

    a_flat = jnp.transpose(lora_A, (1, 0, 2)).reshape(in_dim, er)
    b_flat = lora_B.reshape(er, out_dim)

    TB = 1024
    OB = 512
    grid = (T // TB, out_dim // OB)

    body = functools.partial(_lora_block, rank=rank)
    return pl.pallas_call(
        body,
        grid=grid,
        in_specs=[
            pl.BlockSpec((TB, in_dim), lambda i, j: (i, 0)),
            pl.BlockSpec((TB,), lambda i, j: (i,)),
            pl.BlockSpec((in_dim, er), lambda i, j: (0, 0)),
            pl.BlockSpec((er, OB), lambda i, j: (0, j)),
        ],
        out_specs=pl.BlockSpec((TB, OB), lambda i, j: (i, j)),
        out_shape=jax.ShapeDtypeStruct((T, out_dim), jnp.float32),
        compiler_params=pltpu.CompilerParams(
            dimension_semantics=("parallel", "parallel")),
    )(x, task_id, a_flat, b_flat)


# speedup vs baseline: 1.3959x; 1.3959x over previous
import functools

import jax
import jax.numpy as jnp
from jax.experimental import pallas as pl


def _lora_block(x_ref, tid_ref, a_ref, b_ref, out_ref, *, rank):
    n_cols = a_ref.shape[1]
    tb = x_ref.shape[0]
    tid = jnp.reshape(tid_ref[...], (tb, 1))
    col_expert = jax.lax.broadcasted_iota(jnp.int32, (tb, n_cols), 1) // rank
    xa = jnp.dot(x_ref[...], a_ref[...], preferred_element_type=jnp.float32)
    xa = jnp.where(tid == col_expert, xa, 0.0).astype(jnp.bfloat16)
    out_ref[...] = jnp.dot(xa, b_ref[...].astype(jnp.bfloat16),
                           preferred_element_type=jnp.float32)


def kernel(x, task_id, lora_A, lora_B):
    T, in_dim = x.shape
    n_tasks, _, rank = lora_A.shape
    out_dim = lora_B.shape[2]
    er = n_tasks * rank

    a_flat = jnp.transpose(lora_A, (1, 0, 2)).reshape(in_dim, er)
    b_flat = lora_B.reshape(er, out_dim)

    TB = 1024
    grid = (T // TB,)

    body = functools.partial(_lora_block, rank=rank)
    return pl.pallas_call(
        body,
        grid=grid,
        in_specs=[
            pl.BlockSpec((TB, in_dim), lambda i: (i, 0)),
            pl.BlockSpec((TB,), lambda i: (i,)),
            pl.BlockSpec((in_dim, er), lambda i: (0, 0)),
            pl.BlockSpec((er, out_dim), lambda i: (0, 0)),
        ],
        out_specs=pl.BlockSpec((TB, out_dim), lambda i: (i, 0)),
        out_shape=jax.ShapeDtypeStruct((T, out_dim), jnp.float32),
    )(x, task_id, a_flat, b_flat)


# speedup vs baseline: 1.4581x; 1.0445x over previous
import functools

import jax
import jax.numpy as jnp
from jax.experimental import pallas as pl
from jax.experimental.pallas import tpu as pltpu

_NCHUNK = 8


def _lora_pipe(x_hbm, tid_ref, a_ref, b_ref, out_hbm, xbuf, obuf,
               in_sems, out_sems, *, rank, ch):
    i = pl.program_id(0)
    n = pl.num_programs(0)
    n_cols = a_ref.shape[1]

    @pl.when(i == 0)
    def _():
        for k in range(_NCHUNK):
            pltpu.make_async_copy(
                x_hbm.at[pl.ds(k * ch, ch)], xbuf.at[k], in_sems.at[k]).start()

    pltpu.make_async_copy(
        x_hbm.at[pl.ds(i * ch, ch)], xbuf.at[i], in_sems.at[i]).wait()

    tid = jnp.reshape(tid_ref[pl.ds(i * ch, ch)], (ch, 1))
    col_expert = jax.lax.broadcasted_iota(jnp.int32, (ch, n_cols), 1) // rank
    xa = jnp.dot(xbuf[i], a_ref[...], preferred_element_type=jnp.float32)
    xa = jnp.where(tid == col_expert, xa, 0.0)
    obuf[i] = jnp.dot(xa, b_ref[...], preferred_element_type=jnp.float32)

    pltpu.make_async_copy(
        obuf.at[i], out_hbm.at[pl.ds(i * ch, ch)], out_sems.at[i]).start()

    @pl.when(i == n - 1)
    def _():
        for k in range(_NCHUNK):
            pltpu.make_async_copy(
                obuf.at[k], out_hbm.at[pl.ds(k * ch, ch)], out_sems.at[k]).wait()


def kernel(x, task_id, lora_A, lora_B):
    T, in_dim = x.shape
    n_tasks, _, rank = lora_A.shape
    out_dim = lora_B.shape[2]
    er = n_tasks * rank

    a_flat = jnp.transpose(lora_A, (1, 0, 2)).reshape(in_dim, er)
    b_flat = lora_B.reshape(er, out_dim)

    ch = T // _NCHUNK
    body = functools.partial(_lora_pipe, rank=rank, ch=ch)
    return pl.pallas_call(
        body,
        grid=(_NCHUNK,),
        in_specs=[
            pl.BlockSpec(memory_space=pl.ANY),
            pl.BlockSpec((T,), lambda i: (0,)),
            pl.BlockSpec((in_dim, er), lambda i: (0, 0)),
            pl.BlockSpec((er, out_dim), lambda i: (0, 0)),
        ],
        out_specs=pl.BlockSpec(memory_space=pl.ANY),
        out_shape=jax.ShapeDtypeStruct((T, out_dim), jnp.float32),
        scratch_shapes=[
            pltpu.VMEM((_NCHUNK, T // _NCHUNK, in_dim), jnp.float32),
            pltpu.VMEM((_NCHUNK, T // _NCHUNK, out_dim), jnp.float32),
            pltpu.SemaphoreType.DMA((_NCHUNK,)),
            pltpu.SemaphoreType.DMA((_NCHUNK,)),
        ],
    )(x, task_id, a_flat, b_flat)
